# baseline (speedup 1.0000x reference)
import jax
import jax.numpy as jnp
from jax import lax
from jax.experimental import pallas as pl
from jax.experimental.pallas import tpu as pltpu
from jax.experimental.pallas import tpu_sc as plsc

N_VOX = 500_000
N_HEAD = N_VOX // 128 * 128
N_IN_TILES = N_HEAD // 128
N_TAIL = N_VOX - N_HEAD
N_OUT_TILES = N_VOX * 8 // 128
NW = 32
CH = 1024
IPC = CH // 128
TPC = CH // 16
FULL_CHUNKS = N_HEAD // CH
HEAD_REM_VOX = N_HEAD - FULL_CHUNKS * CH
BASE_ITERS = FULL_CHUNKS // NW
EXTRA_W = FULL_CHUNKS % NW

_OFF_BITS = (210, 180, 232, 0)
_MULS = (2, 2, 2, 1)


def _tec_body(xh_hbm, xt_hbm, out_hbm, in_v, in_t, out_v, sem_in, sem_out):
    nc = 2
    w = lax.axis_index("s") * nc + lax.axis_index("c")

    lane = lax.iota(jnp.int32, 16)
    duo = lane // 8
    corner = lane % 8
    offv = [((b >> corner) & 1).astype(jnp.int32) for b in _OFF_BITS[:3]]
    jvec = [jnp.broadcast_to(jnp.int32(j), (16,)) for j in range(4)]

    nits = jnp.where(w < EXTRA_W, BASE_ITERS + 1, BASE_ITERS)

    def start_in(i, buf):
        cid = w + NW * i
        pltpu.make_async_copy(
            xh_hbm.at[pl.ds(cid * IPC, IPC)],
            in_v.at[pl.ds(buf * IPC, IPC)], sem_in).start()

    def wait_in():
        pltpu.make_async_copy(
            xh_hbm.at[pl.ds(0, IPC)], in_v.at[pl.ds(0, IPC)], sem_in).wait()

    def wait_out():
        pltpu.make_async_copy(
            out_v.at[pl.ds(0, TPC)],
            out_hbm.at[pl.ds(0, TPC)], sem_out).wait()

    idxs = [lane * 8 + s for s in range(8)]

    def compute_chunk(buf, cid):
        ib = buf * IPC
        ob = buf * TPC

        for bt in range(IPC):
            t = ib + bt
            og0 = ob + 8 * bt
            for gg in range(8):
                og = og0 + gg
                isl = pl.ds(16 * gg, 16)
                for j in range(4):
                    src = in_v[t, j, isl]
                    if j < 3:
                        src = src + src
                        srcp1 = src + 1
                    else:
                        srcp1 = src
                    for s in range(8):
                        data = srcp1 if (_OFF_BITS[j] >> s) & 1 else src
                        plsc.store_scatter(out_v.at[og, j], [idxs[s]], data)
            pltpu.make_async_copy(
                out_v.at[pl.ds(og0, 8)],
                out_hbm.at[pl.ds(cid * TPC + 8 * bt, 8)], sem_out).start()

    start_in(0, 0)
    start_in(1, 1)

    def loop_body(i, carry):
        buf = i % 3
        wait_in()

        @pl.when(i + 2 < nits)
        def _():
            start_in(i + 2, (i + 2) % 3)

        @pl.when(i >= 3)
        def _():
            wait_out()

        compute_chunk(buf, w + NW * i)
        return carry
    lax.fori_loop(0, nits, loop_body, 0)

    wait_out()
    wait_out()
    wait_out()

    @pl.when(w == EXTRA_W)
    def _():
        pltpu.sync_copy(xh_hbm.at[pl.ds(FULL_CHUNKS * IPC, 2)],
                        in_v.at[pl.ds(0, 2)])
        for bt in range(2):
            for j in range(3):
                for v in range(8):
                    sl = pl.ds(16 * v, 16)
                    in_v[bt, j, sl] = in_v[bt, j, sl] * 2
            for gg in range(8):
                for s in range(8):
                    lv = duo + (16 * gg + 2 * s)
                    osl = pl.ds(16 * s, 16)
                    for j in range(3):
                        gj = plsc.load_gather(in_v.at[bt, j], [lv])
                        out_v[8 * bt + gg, j, osl] = gj + offv[j]
                    g3 = plsc.load_gather(in_v.at[bt, 3], [lv])
                    out_v[8 * bt + gg, 3, osl] = g3
        pltpu.sync_copy(xt_hbm, in_t)
        for g in range(2):
            for s in range(8):
                rows = 16 * g + 2 * s + duo
                osl = pl.ds(16 * s, 16)
                for j in range(3):
                    gj = plsc.load_gather(in_t, [rows, jvec[j]])
                    out_v[16 + g, j, osl] = gj * 2 + offv[j]
                g3 = plsc.load_gather(in_t, [rows, jvec[3]])
                out_v[16 + g, 3, osl] = g3
        pltpu.sync_copy(out_v.at[pl.ds(0, 18)],
                        out_hbm.at[pl.ds(FULL_CHUNKS * TPC, 18)])


def kernel(voxel_inds):
    xh = voxel_inds[:N_HEAD].reshape(N_IN_TILES, 128, 4).transpose(0, 2, 1)
    xt = voxel_inds[N_HEAD:]
    mesh = plsc.VectorSubcoreMesh(core_axis_name="c", subcore_axis_name="s")
    out3 = pl.kernel(
        _tec_body,
        out_type=jax.ShapeDtypeStruct((N_OUT_TILES, 4, 128), jnp.int32),
        mesh=mesh,
        compiler_params=pltpu.CompilerParams(
            needs_layout_passes=False, use_tc_tiling_on_sc=False),
        scratch_types=[
            pltpu.VMEM((3 * IPC, 4, 128), jnp.int32),
            pltpu.VMEM((N_TAIL, 4), jnp.int32),
            pltpu.VMEM((3 * TPC, 4, 128), jnp.int32),
            pltpu.SemaphoreType.DMA,
            pltpu.SemaphoreType.DMA,
        ],
    )(xh, xt)
    return out3.transpose(0, 2, 1).reshape(-1, 4)


# speedup vs baseline: 1.4119x; 1.4119x over previous
import jax
import jax.numpy as jnp
from jax import lax
from jax.experimental import pallas as pl
from jax.experimental.pallas import tpu as pltpu
from jax.experimental.pallas import tpu_sc as plsc

N_VOX = 500_000
N_HEAD = N_VOX // 128 * 128
N_IN_TILES = N_HEAD // 128
N_TAIL = N_VOX - N_HEAD
N_OUT_TILES = N_VOX * 8 // 128
NW = 32
CH = 1024
IPC = CH // 128
TPC = CH // 16
FULL_CHUNKS = N_HEAD // CH
HEAD_REM_VOX = N_HEAD - FULL_CHUNKS * CH
BASE_ITERS = FULL_CHUNKS // NW
EXTRA_W = FULL_CHUNKS % NW

_OFF_BITS = (210, 180, 232, 0)
_MULS = (2, 2, 2, 1)


def _tec_body(xh_hbm, xt_hbm, out_hbm, in_v, in_t, out_v, sem_in, sem_out):
    nc = 2
    w = lax.axis_index("s") * nc + lax.axis_index("c")

    lane = lax.iota(jnp.int32, 16)
    duo = lane // 8
    corner = lane % 8
    offv = [((b >> corner) & 1).astype(jnp.int32) for b in _OFF_BITS[:3]]
    jvec = [jnp.broadcast_to(jnp.int32(j), (16,)) for j in range(4)]

    nits = jnp.where(w < EXTRA_W, BASE_ITERS + 1, BASE_ITERS)

    def start_in(i, buf):
        cid = w + NW * i
        pltpu.make_async_copy(
            xh_hbm.at[pl.ds(cid * IPC, IPC)],
            in_v.at[pl.ds(buf * IPC, IPC)], sem_in).start()

    def wait_in():
        pltpu.make_async_copy(
            xh_hbm.at[pl.ds(0, IPC)], in_v.at[pl.ds(0, IPC)], sem_in).wait()

    def wait_out():
        pltpu.make_async_copy(
            out_v.at[pl.ds(0, TPC)],
            out_hbm.at[pl.ds(0, TPC)], sem_out).wait()

    idxs = [lane * 8 + s for s in range(8)]

    def compute_chunk(buf, cid):
        ib = buf * IPC
        ob = buf * TPC

        def blk(bt, carry):
            t = ib + bt
            og0 = ob + 8 * bt
            for gg in range(8):
                og = og0 + gg
                isl = pl.ds(16 * gg, 16)
                for j in range(4):
                    src = in_v[t, j, isl]
                    if j < 3:
                        src = src + src
                        srcp1 = src + 1
                    else:
                        srcp1 = src
                    for s in range(8):
                        data = srcp1 if (_OFF_BITS[j] >> s) & 1 else src
                        plsc.store_scatter(out_v.at[og, j], [idxs[s]], data)
            pltpu.make_async_copy(
                out_v.at[pl.ds(og0, 8)],
                out_hbm.at[pl.ds(cid * TPC + 8 * bt, 8)], sem_out).start()
            return carry
        lax.fori_loop(0, IPC, blk, 0)

    start_in(0, 0)
    start_in(1, 1)

    def loop_body(i, carry):
        buf = i % 3
        wait_in()

        @pl.when(i + 2 < nits)
        def _():
            start_in(i + 2, (i + 2) % 3)

        @pl.when(i >= 3)
        def _():
            wait_out()

        compute_chunk(buf, w + NW * i)
        return carry
    lax.fori_loop(0, nits, loop_body, 0)

    wait_out()
    wait_out()
    wait_out()

    @pl.when(w == EXTRA_W)
    def _():
        pltpu.sync_copy(xh_hbm.at[pl.ds(FULL_CHUNKS * IPC, 2)],
                        in_v.at[pl.ds(0, 2)])
        for bt in range(2):
            for j in range(3):
                for v in range(8):
                    sl = pl.ds(16 * v, 16)
                    in_v[bt, j, sl] = in_v[bt, j, sl] * 2
            for gg in range(8):
                for s in range(8):
                    lv = duo + (16 * gg + 2 * s)
                    osl = pl.ds(16 * s, 16)
                    for j in range(3):
                        gj = plsc.load_gather(in_v.at[bt, j], [lv])
                        out_v[8 * bt + gg, j, osl] = gj + offv[j]
                    g3 = plsc.load_gather(in_v.at[bt, 3], [lv])
                    out_v[8 * bt + gg, 3, osl] = g3
        pltpu.sync_copy(xt_hbm, in_t)
        for g in range(2):
            for s in range(8):
                rows = 16 * g + 2 * s + duo
                osl = pl.ds(16 * s, 16)
                for j in range(3):
                    gj = plsc.load_gather(in_t, [rows, jvec[j]])
                    out_v[16 + g, j, osl] = gj * 2 + offv[j]
                g3 = plsc.load_gather(in_t, [rows, jvec[3]])
                out_v[16 + g, 3, osl] = g3
        pltpu.sync_copy(out_v.at[pl.ds(0, 18)],
                        out_hbm.at[pl.ds(FULL_CHUNKS * TPC, 18)])


def kernel(voxel_inds):
    xh = voxel_inds[:N_HEAD].reshape(N_IN_TILES, 128, 4).transpose(0, 2, 1)
    xt = voxel_inds[N_HEAD:]
    mesh = plsc.VectorSubcoreMesh(core_axis_name="c", subcore_axis_name="s")
    out3 = pl.kernel(
        _tec_body,
        out_type=jax.ShapeDtypeStruct((N_OUT_TILES, 4, 128), jnp.int32),
        mesh=mesh,
        compiler_params=pltpu.CompilerParams(
            needs_layout_passes=False, use_tc_tiling_on_sc=False),
        scratch_types=[
            pltpu.VMEM((3 * IPC, 4, 128), jnp.int32),
            pltpu.VMEM((N_TAIL, 4), jnp.int32),
            pltpu.VMEM((3 * TPC, 4, 128), jnp.int32),
            pltpu.SemaphoreType.DMA,
            pltpu.SemaphoreType.DMA,
        ],
    )(xh, xt)
    return out3.transpose(0, 2, 1).reshape(-1, 4)
